# baseline (speedup 1.0000x reference)
import functools

import jax
import jax.numpy as jnp
from jax import lax
from jax.experimental import pallas as pl
from jax.experimental.pallas import tpu as pltpu
from jax.experimental.pallas import tpu_sc as plsc

N = 10000
E = 320000
D = 128

NC = 2
NS = 16
NW = NC * NS
EPW = E // NW
CH = 80
NCHUNK = EPW // CH
N2 = 10240
RPT = N2 // NS
ZR = 40


CH2 = 128
NFULL = EPW // CH2
TAIL = EPW - NFULL * CH2
NIB = 4
ZB = 64


def _make_agg():
    mesh = plsc.VectorSubcoreMesh(core_axis_name="c", subcore_axis_name="s")
    out_type = jax.ShapeDtypeStruct((NC, N2, D), jnp.float32)
    scratch = [
        pltpu.VMEM((NIB, CH2), jnp.int32),
        pltpu.VMEM((NIB, CH2), jnp.int32),
        pltpu.VMEM((2, CH2, D), jnp.float32),
        pltpu.VMEM((TAIL,), jnp.int32),
        pltpu.VMEM((TAIL,), jnp.int32),
        pltpu.VMEM((TAIL, D), jnp.float32),
        pltpu.VMEM((ZB, D), jnp.float32),
        pltpu.VMEM_SHARED((N2, D), jnp.float32),
    ] + [pltpu.SemaphoreType.DMA] * (NIB + 8 + 2)

    @functools.partial(
        pl.kernel, mesh=mesh, out_type=out_type, scratch_types=scratch)
    def agg(row_hbm, col_hbm, tab_hbm, out_hbm, rowv, colv, rows,
            trowv, tcolv, trows, zbuf, acc, *sems):
        semL = sems[:NIB]
        semG = sems[NIB:NIB + 8]
        semS = sems[NIB + 8:]
        cid = lax.axis_index("c")
        sid = lax.axis_index("s")
        base = (cid * NS + sid) * EPW

        def load_idx(j, m):
            off = base + j * CH2
            pltpu.async_copy(row_hbm.at[pl.ds(off, CH2)], rowv.at[m], semL[m])
            pltpu.async_copy(col_hbm.at[pl.ds(off, CH2)], colv.at[m], semL[m])

        def wait_idx(m):
            pltpu.make_async_copy(row_hbm.at[pl.ds(0, CH2)], rowv.at[m],
                                  semL[m]).wait()
            pltpu.make_async_copy(col_hbm.at[pl.ds(0, CH2)], colv.at[m],
                                  semL[m]).wait()

        H = CH2 // 4

        def gather(m, b):
            for q in range(4):
                pltpu.async_copy(tab_hbm.at[colv.at[m, pl.ds(q * H, H)]],
                                 rows.at[b, pl.ds(q * H, H)], semG[4 * b + q])

        def wait_gather(b):
            for q in range(4):
                pltpu.make_async_copy(tab_hbm.at[colv.at[0, pl.ds(q * H, H)]],
                                      rows.at[b, pl.ds(q * H, H)],
                                      semG[4 * b + q]).wait()

        def scatter(m, b):
            pltpu.async_copy(rows.at[b], acc.at[rowv.at[m]], semS[b],
                             add=True)

        def wait_scatter(b):
            pltpu.make_async_copy(rows.at[b], acc.at[rowv.at[0]],
                                  semS[b]).wait()

        load_idx(0, 0)
        load_idx(1, 1)
        load_idx(2, 2)

        def fill(i, _):
            for j in range(D // 16):
                zbuf[i, pl.ds(j * 16, 16)] = jnp.zeros((16,), jnp.float32)
            return ()
        lax.fori_loop(0, ZB, fill, ())

        def zero(i, _):
            pltpu.sync_copy(zbuf, acc.at[pl.ds(sid * RPT + i * ZB, ZB)])
            return ()
        lax.fori_loop(0, RPT // ZB, zero, ())
        plsc.subcore_barrier()

        wait_idx(0)
        gather(0, 0)
        wait_gather(0)
        scatter(0, 0)
        wait_idx(1)
        gather(1, 1)
        load_idx(3, 3)

        def body(k, _):
            j = 1 + 4 * k
            for t in range(4):
                m = (1 + t) % NIB
                b = (1 + t) % 2
                nm = (m + 1) % NIB
                nb = 1 - b
                wait_gather(b)
                scatter(m, b)
                wait_scatter(nb)
                wait_idx(nm)
                gather(nm, nb)
                load_idx(j + t + 3, (m + 3) % NIB)
            return ()
        lax.fori_loop(0, 18, body, ())

        for j in range(73, NFULL):
            m = j % NIB
            b = j % 2
            nm = (m + 1) % NIB
            nb = 1 - b
            wait_gather(b)
            scatter(m, b)
            if j + 1 < NFULL:
                wait_scatter(nb)
                wait_idx(nm)
                gather(nm, nb)
                if j + 3 < NFULL:
                    load_idx(j + 3, (m + 3) % NIB)
        wait_scatter(0)
        wait_scatter(1)
        toff = base + NFULL * CH2
        pltpu.sync_copy(row_hbm.at[pl.ds(toff, TAIL)], trowv)
        pltpu.sync_copy(col_hbm.at[pl.ds(toff, TAIL)], tcolv)
        pltpu.async_copy(tab_hbm.at[tcolv], trows, semG[0]).wait()
        pltpu.sync_copy(trows, acc.at[trowv], add=True)
        plsc.subcore_barrier()

        r0 = sid * RPT
        pltpu.sync_copy(acc.at[pl.ds(r0, RPT)],
                        out_hbm.at[cid, pl.ds(r0, RPT)])

    return agg


def _make_deg():
    mesh = plsc.VectorSubcoreMesh(core_axis_name="c", subcore_axis_name="s")
    out_type = jax.ShapeDtypeStruct((NC, N2, D), jnp.float32)
    scratch = [
        pltpu.VMEM((NIB, CH2), jnp.int32),
        pltpu.VMEM((TAIL,), jnp.int32),
        pltpu.VMEM((CH2, D), jnp.float32),
        pltpu.VMEM((ZB, D), jnp.float32),
        pltpu.VMEM_SHARED((N2, D), jnp.float32),
    ] + [pltpu.SemaphoreType.DMA] * (NIB + 2)

    @functools.partial(
        pl.kernel, mesh=mesh, out_type=out_type, scratch_types=scratch)
    def deg(row_hbm, out_hbm, rowv, trowv, obuf, zbuf, acc, *sems):
        semL = sems[:NIB]
        semS = sems[NIB:]
        cid = lax.axis_index("c")
        sid = lax.axis_index("s")
        base = (cid * NS + sid) * EPW

        def load_idx(j, m):
            pltpu.async_copy(row_hbm.at[pl.ds(base + j * CH2, CH2)],
                             rowv.at[m], semL[m])

        def wait_idx(m):
            pltpu.make_async_copy(row_hbm.at[pl.ds(0, CH2)], rowv.at[m],
                                  semL[m]).wait()

        def scatter(m, b):
            pltpu.async_copy(obuf, acc.at[rowv.at[m]], semS[b], add=True)

        def wait_scatter(b):
            pltpu.make_async_copy(obuf, acc.at[rowv.at[0]], semS[b]).wait()

        load_idx(0, 0)
        load_idx(1, 1)
        load_idx(2, 2)

        def fill(i, _):
            for j in range(D // 16):
                zbuf[i, pl.ds(j * 16, 16)] = jnp.zeros((16,), jnp.float32)
            return ()
        lax.fori_loop(0, ZB, fill, ())

        def fillo(i, _):
            for j in range(D // 16):
                obuf[i, pl.ds(j * 16, 16)] = jnp.ones((16,), jnp.float32)
            return ()
        lax.fori_loop(0, CH2, fillo, ())

        def zero(i, _):
            pltpu.sync_copy(zbuf, acc.at[pl.ds(sid * RPT + i * ZB, ZB)])
            return ()
        lax.fori_loop(0, RPT // ZB, zero, ())
        plsc.subcore_barrier()

        wait_idx(0)
        scatter(0, 0)
        wait_idx(1)
        scatter(1, 1)
        load_idx(3, 3)

        def body(k, _):
            j = 2 + 4 * k
            for t in range(4):
                m = (2 + t) % NIB
                b = (2 + t) % 2
                wait_scatter(b)
                wait_idx(m)
                scatter(m, b)
                load_idx(j + t + 2, (m + 2) % NIB)
            return ()
        lax.fori_loop(0, 18, body, ())

        for j in range(74, NFULL):
            m = j % NIB
            b = j % 2
            wait_scatter(b)
            wait_idx(m)
            scatter(m, b)
            if j + 2 < NFULL:
                load_idx(j + 2, (m + 2) % NIB)
        wait_scatter(0)
        wait_scatter(1)
        toff = base + NFULL * CH2
        pltpu.sync_copy(row_hbm.at[pl.ds(toff, TAIL)], trowv)
        pltpu.sync_copy(obuf.at[pl.ds(0, TAIL)], acc.at[trowv], add=True)
        plsc.subcore_barrier()

        r0 = sid * RPT
        pltpu.sync_copy(acc.at[pl.ds(r0, RPT)],
                        out_hbm.at[cid, pl.ds(r0, RPT)])

    return deg


_agg = _make_agg()
_deg = _make_deg()

BR = 1000


def _mm_body(x_ref, w_ref, o_ref):
    o_ref[...] = jnp.dot(x_ref[...], w_ref[...],
                         preferred_element_type=jnp.float32)


_mm = pl.pallas_call(
    _mm_body,
    grid=(N // BR,),
    in_specs=[
        pl.BlockSpec((BR, D), lambda i: (i, 0)),
        pl.BlockSpec((D, D), lambda i: (0, 0)),
    ],
    out_specs=pl.BlockSpec((BR, D), lambda i: (i, 0)),
    out_shape=jax.ShapeDtypeStruct((N, D), jnp.float32),
)


def _mid_body(y1_ref, p_ref, dp_ref, b1_ref, w2_ref, y2_ref, inv_ref):
    s = p_ref[0] + p_ref[1]
    deg = dp_ref[0][:, 0:1] + dp_ref[1][:, 0:1]
    inv = jnp.where(deg > 0, 1.0 / deg, 0.0)
    h = inv * s + y1_ref[...] + b1_ref[...]
    h = jnp.where(h >= 0, h, 0.2 * h)
    y2_ref[...] = jnp.dot(h, w2_ref[...], preferred_element_type=jnp.float32)
    inv_ref[...] = jnp.broadcast_to(inv, (BR, D))


_mid = pl.pallas_call(
    _mid_body,
    grid=(N // BR,),
    in_specs=[
        pl.BlockSpec((BR, D), lambda i: (i, 0)),
        pl.BlockSpec((NC, BR, D), lambda i: (0, i, 0)),
        pl.BlockSpec((NC, BR, D), lambda i: (0, i, 0)),
        pl.BlockSpec((1, D), lambda i: (0, 0)),
        pl.BlockSpec((D, D), lambda i: (0, 0)),
    ],
    out_specs=[
        pl.BlockSpec((BR, D), lambda i: (i, 0)),
        pl.BlockSpec((BR, D), lambda i: (i, 0)),
    ],
    out_shape=[
        jax.ShapeDtypeStruct((N, D), jnp.float32),
        jax.ShapeDtypeStruct((N, D), jnp.float32),
    ],
)


def _final_body(y2_ref, q_ref, inv_ref, b2_ref, o_ref):
    s = q_ref[0] + q_ref[1]
    h = inv_ref[...] * s + y2_ref[...] + b2_ref[...]
    n2 = jnp.sum(h * h, axis=1, keepdims=True)
    norm = jnp.sqrt(n2)
    o_ref[...] = h / jnp.maximum(norm, 1e-12)


_final = pl.pallas_call(
    _final_body,
    grid=(N // BR,),
    in_specs=[
        pl.BlockSpec((BR, D), lambda i: (i, 0)),
        pl.BlockSpec((NC, BR, D), lambda i: (0, i, 0)),
        pl.BlockSpec((BR, D), lambda i: (i, 0)),
        pl.BlockSpec((1, D), lambda i: (0, 0)),
    ],
    out_specs=pl.BlockSpec((BR, D), lambda i: (i, 0)),
    out_shape=jax.ShapeDtypeStruct((N, D), jnp.float32),
)


def kernel(x, edge_index, W1, b1, W2, b2):
    row = edge_index[0]
    col = edge_index[1]
    y1 = _mm(x, W1)
    degp = _deg(row)
    part = _agg(row, col, y1)
    y2, invb = _mid(y1, part, degp, b1.reshape(1, D), W2)
    qpart = _agg(row, col, y2)
    out = _final(y2, qpart, invb, b2.reshape(1, D))
    return out


# speedup vs baseline: 1.0221x; 1.0221x over previous
import functools

import jax
import jax.numpy as jnp
from jax import lax
from jax.experimental import pallas as pl
from jax.experimental.pallas import tpu as pltpu
from jax.experimental.pallas import tpu_sc as plsc

N = 10000
E = 320000
D = 128

NC = 2
NS = 16
NW = NC * NS
EPW = E // NW
CH = 80
NCHUNK = EPW // CH
N2 = 10240
RPT = N2 // NS
ZR = 40


CH2 = 128
NFULL = EPW // CH2
TAIL = EPW - NFULL * CH2
NIB = 4
ZB = 64


def _make_agg():
    mesh = plsc.VectorSubcoreMesh(core_axis_name="c", subcore_axis_name="s")
    out_type = jax.ShapeDtypeStruct((NC, N2, D), jnp.float32)
    scratch = [
        pltpu.VMEM((NIB, CH2), jnp.int32),
        pltpu.VMEM((NIB, CH2), jnp.int32),
        pltpu.VMEM((2, CH2, D), jnp.float32),
        pltpu.VMEM((TAIL,), jnp.int32),
        pltpu.VMEM((TAIL,), jnp.int32),
        pltpu.VMEM((TAIL, D), jnp.float32),
        pltpu.VMEM((ZB, D), jnp.float32),
        pltpu.VMEM_SHARED((N2, D), jnp.float32),
    ] + [pltpu.SemaphoreType.DMA] * (NIB + 4 + 2)

    @functools.partial(
        pl.kernel, mesh=mesh, out_type=out_type, scratch_types=scratch)
    def agg(row_hbm, col_hbm, tab_hbm, out_hbm, rowv, colv, rows,
            trowv, tcolv, trows, zbuf, acc, *sems):
        semL = sems[:NIB]
        semG = sems[NIB:NIB + 4]
        semS = sems[NIB + 4:]
        cid = lax.axis_index("c")
        sid = lax.axis_index("s")
        base = (cid * NS + sid) * EPW

        def load_idx(j, m):
            off = base + j * CH2
            pltpu.async_copy(row_hbm.at[pl.ds(off, CH2)], rowv.at[m], semL[m])
            pltpu.async_copy(col_hbm.at[pl.ds(off, CH2)], colv.at[m], semL[m])

        def wait_idx(m):
            pltpu.make_async_copy(row_hbm.at[pl.ds(0, CH2)], rowv.at[m],
                                  semL[m]).wait()
            pltpu.make_async_copy(col_hbm.at[pl.ds(0, CH2)], colv.at[m],
                                  semL[m]).wait()

        H = CH2 // 2

        def gather(m, b):
            pltpu.async_copy(tab_hbm.at[colv.at[m, pl.ds(0, H)]],
                             rows.at[b, pl.ds(0, H)], semG[2 * b])
            pltpu.async_copy(tab_hbm.at[colv.at[m, pl.ds(H, H)]],
                             rows.at[b, pl.ds(H, H)], semG[2 * b + 1])

        def wait_gather(b):
            pltpu.make_async_copy(tab_hbm.at[colv.at[0, pl.ds(0, H)]],
                                  rows.at[b, pl.ds(0, H)], semG[2 * b]).wait()
            pltpu.make_async_copy(tab_hbm.at[colv.at[0, pl.ds(H, H)]],
                                  rows.at[b, pl.ds(H, H)],
                                  semG[2 * b + 1]).wait()

        def scatter(m, b):
            pltpu.async_copy(rows.at[b], acc.at[rowv.at[m]], semS[b],
                             add=True)

        def wait_scatter(b):
            pltpu.make_async_copy(rows.at[b], acc.at[rowv.at[0]],
                                  semS[b]).wait()

        load_idx(0, 0)
        load_idx(1, 1)
        load_idx(2, 2)

        def fill(i, _):
            for j in range(D // 16):
                zbuf[i, pl.ds(j * 16, 16)] = jnp.zeros((16,), jnp.float32)
            return ()
        lax.fori_loop(0, ZB, fill, ())

        def zero(i, _):
            pltpu.sync_copy(zbuf, acc.at[pl.ds(sid * RPT + i * ZB, ZB)])
            return ()
        lax.fori_loop(0, RPT // ZB, zero, ())
        plsc.subcore_barrier()

        wait_idx(0)
        gather(0, 0)
        wait_gather(0)
        scatter(0, 0)
        wait_idx(1)
        gather(1, 1)
        load_idx(3, 3)

        def body(k, _):
            j = 1 + 4 * k
            for t in range(4):
                m = (1 + t) % NIB
                b = (1 + t) % 2
                nm = (m + 1) % NIB
                nb = 1 - b
                wait_gather(b)
                scatter(m, b)
                wait_scatter(nb)
                wait_idx(nm)
                gather(nm, nb)
                load_idx(j + t + 3, (m + 3) % NIB)
            return ()
        lax.fori_loop(0, 18, body, ())

        for j in range(73, NFULL):
            m = j % NIB
            b = j % 2
            nm = (m + 1) % NIB
            nb = 1 - b
            wait_gather(b)
            scatter(m, b)
            if j + 1 < NFULL:
                wait_scatter(nb)
                wait_idx(nm)
                gather(nm, nb)
                if j + 3 < NFULL:
                    load_idx(j + 3, (m + 3) % NIB)
        wait_scatter(0)
        wait_scatter(1)
        toff = base + NFULL * CH2
        pltpu.sync_copy(row_hbm.at[pl.ds(toff, TAIL)], trowv)
        pltpu.sync_copy(col_hbm.at[pl.ds(toff, TAIL)], tcolv)
        pltpu.async_copy(tab_hbm.at[tcolv], trows, semG[0]).wait()
        pltpu.sync_copy(trows, acc.at[trowv], add=True)
        plsc.subcore_barrier()

        r0 = sid * RPT
        pltpu.sync_copy(acc.at[pl.ds(r0, RPT)],
                        out_hbm.at[cid, pl.ds(r0, RPT)])

    return agg


def _make_deg_agg():
    mesh = plsc.VectorSubcoreMesh(core_axis_name="c", subcore_axis_name="s")
    out_type = [
        jax.ShapeDtypeStruct((NC, N2, D), jnp.float32),
        jax.ShapeDtypeStruct((NC, N2, D), jnp.float32),
    ]
    scratch = [
        pltpu.VMEM((NIB, CH2), jnp.int32),
        pltpu.VMEM((NIB, CH2), jnp.int32),
        pltpu.VMEM((2, CH2, D), jnp.float32),
        pltpu.VMEM((TAIL,), jnp.int32),
        pltpu.VMEM((TAIL,), jnp.int32),
        pltpu.VMEM((TAIL, D), jnp.float32),
        pltpu.VMEM((ZB, D), jnp.float32),
        pltpu.VMEM_SHARED((N2, D), jnp.float32),
    ] + [pltpu.SemaphoreType.DMA] * (NIB + 4 + 2)

    @functools.partial(
        pl.kernel, mesh=mesh, out_type=out_type, scratch_types=scratch)
    def degagg(row_hbm, col_hbm, tab_hbm, deg_hbm, out_hbm, rowv, colv,
               rows, trowv, tcolv, trows, zbuf, acc, *sems):
        semL = sems[:NIB]
        semG = sems[NIB:NIB + 4]
        semS = sems[NIB + 4:]
        cid = lax.axis_index("c")
        sid = lax.axis_index("s")
        base = (cid * NS + sid) * EPW

        def load_idx(j, m):
            off = base + j * CH2
            pltpu.async_copy(row_hbm.at[pl.ds(off, CH2)], rowv.at[m], semL[m])
            pltpu.async_copy(col_hbm.at[pl.ds(off, CH2)], colv.at[m], semL[m])

        def wait_idx(m):
            pltpu.make_async_copy(row_hbm.at[pl.ds(0, CH2)], rowv.at[m],
                                  semL[m]).wait()
            pltpu.make_async_copy(col_hbm.at[pl.ds(0, CH2)], colv.at[m],
                                  semL[m]).wait()

        H = CH2 // 2

        def gather(m, b):
            pltpu.async_copy(tab_hbm.at[colv.at[m, pl.ds(0, H)]],
                             rows.at[b, pl.ds(0, H)], semG[2 * b])
            pltpu.async_copy(tab_hbm.at[colv.at[m, pl.ds(H, H)]],
                             rows.at[b, pl.ds(H, H)], semG[2 * b + 1])

        def wait_gather(b):
            pltpu.make_async_copy(tab_hbm.at[colv.at[0, pl.ds(0, H)]],
                                  rows.at[b, pl.ds(0, H)], semG[2 * b]).wait()
            pltpu.make_async_copy(tab_hbm.at[colv.at[0, pl.ds(H, H)]],
                                  rows.at[b, pl.ds(H, H)],
                                  semG[2 * b + 1]).wait()

        def scatter(m, b):
            pltpu.async_copy(rows.at[b], acc.at[rowv.at[m]], semS[b],
                             add=True)

        def wait_scatter(b):
            pltpu.make_async_copy(rows.at[b], acc.at[rowv.at[0]],
                                  semS[b]).wait()

        def dscatter(m, b):
            pltpu.async_copy(rows.at[0], acc.at[rowv.at[m]], semS[b],
                             add=True)

        def wait_dscatter(b):
            pltpu.make_async_copy(rows.at[0], acc.at[rowv.at[0]],
                                  semS[b]).wait()

        def zero_own_slice():
            def zero(i, _):
                pltpu.sync_copy(zbuf,
                                acc.at[pl.ds(sid * RPT + i * ZB, ZB)])
                return ()
            lax.fori_loop(0, RPT // ZB, zero, ())

        load_idx(0, 0)
        load_idx(1, 1)
        load_idx(2, 2)

        def fill(i, _):
            for j in range(D // 16):
                zbuf[i, pl.ds(j * 16, 16)] = jnp.zeros((16,), jnp.float32)
            return ()
        lax.fori_loop(0, ZB, fill, ())

        def fillo(i, _):
            for j in range(D // 16):
                rows[0, i, pl.ds(j * 16, 16)] = jnp.ones((16,), jnp.float32)
            return ()
        lax.fori_loop(0, CH2, fillo, ())

        zero_own_slice()
        plsc.subcore_barrier()

        wait_idx(0)
        dscatter(0, 0)
        wait_idx(1)
        dscatter(1, 1)
        load_idx(3, 3)

        def dbody(k, _):
            j = 2 + 4 * k
            for t in range(4):
                m = (2 + t) % NIB
                b = (2 + t) % 2
                wait_dscatter(b)
                wait_idx(m)
                dscatter(m, b)
                load_idx(j + t + 2, (m + 2) % NIB)
            return ()
        lax.fori_loop(0, 18, dbody, ())

        for j in range(74, NFULL):
            m = j % NIB
            b = j % 2
            wait_dscatter(b)
            wait_idx(m)
            dscatter(m, b)
            if j + 2 < NFULL:
                load_idx(j + 2, (m + 2) % NIB)
        wait_dscatter(0)
        wait_dscatter(1)
        toff = base + NFULL * CH2
        pltpu.sync_copy(row_hbm.at[pl.ds(toff, TAIL)], trowv)
        pltpu.sync_copy(rows.at[0, pl.ds(0, TAIL)], acc.at[trowv], add=True)
        plsc.subcore_barrier()

        r0 = sid * RPT
        pltpu.sync_copy(acc.at[pl.ds(r0, RPT)],
                        deg_hbm.at[cid, pl.ds(r0, RPT)])
        zero_own_slice()
        load_idx(0, 0)
        load_idx(1, 1)
        load_idx(2, 2)
        plsc.subcore_barrier()

        wait_idx(0)
        gather(0, 0)
        wait_gather(0)
        scatter(0, 0)
        wait_idx(1)
        gather(1, 1)
        load_idx(3, 3)

        def body(k, _):
            j = 1 + 4 * k
            for t in range(4):
                m = (1 + t) % NIB
                b = (1 + t) % 2
                nm = (m + 1) % NIB
                nb = 1 - b
                wait_gather(b)
                scatter(m, b)
                wait_scatter(nb)
                wait_idx(nm)
                gather(nm, nb)
                load_idx(j + t + 3, (m + 3) % NIB)
            return ()
        lax.fori_loop(0, 18, body, ())

        for j in range(73, NFULL):
            m = j % NIB
            b = j % 2
            nm = (m + 1) % NIB
            nb = 1 - b
            wait_gather(b)
            scatter(m, b)
            if j + 1 < NFULL:
                wait_scatter(nb)
                wait_idx(nm)
                gather(nm, nb)
                if j + 3 < NFULL:
                    load_idx(j + 3, (m + 3) % NIB)
        wait_scatter(0)
        wait_scatter(1)
        pltpu.sync_copy(row_hbm.at[pl.ds(toff, TAIL)], trowv)
        pltpu.sync_copy(col_hbm.at[pl.ds(toff, TAIL)], tcolv)
        pltpu.async_copy(tab_hbm.at[tcolv], trows, semG[0]).wait()
        pltpu.sync_copy(trows, acc.at[trowv], add=True)
        plsc.subcore_barrier()

        pltpu.sync_copy(acc.at[pl.ds(r0, RPT)],
                        out_hbm.at[cid, pl.ds(r0, RPT)])

    return degagg


_agg = _make_agg()
_degagg = _make_deg_agg()

BR = 1000


def _mm_body(x_ref, w_ref, o_ref):
    o_ref[...] = jnp.dot(x_ref[...], w_ref[...],
                         preferred_element_type=jnp.float32)


_mm = pl.pallas_call(
    _mm_body,
    grid=(N // BR,),
    in_specs=[
        pl.BlockSpec((BR, D), lambda i: (i, 0)),
        pl.BlockSpec((D, D), lambda i: (0, 0)),
    ],
    out_specs=pl.BlockSpec((BR, D), lambda i: (i, 0)),
    out_shape=jax.ShapeDtypeStruct((N, D), jnp.float32),
)


def _mid_body(y1_ref, p_ref, dp_ref, b1_ref, w2_ref, y2_ref, inv_ref):
    s = p_ref[0] + p_ref[1]
    deg = dp_ref[0][:, 0:1] + dp_ref[1][:, 0:1]
    inv = jnp.where(deg > 0, 1.0 / deg, 0.0)
    h = inv * s + y1_ref[...] + b1_ref[...]
    h = jnp.where(h >= 0, h, 0.2 * h)
    y2_ref[...] = jnp.dot(h, w2_ref[...], preferred_element_type=jnp.float32)
    inv_ref[...] = jnp.broadcast_to(inv, (BR, D))


_mid = pl.pallas_call(
    _mid_body,
    grid=(N // BR,),
    in_specs=[
        pl.BlockSpec((BR, D), lambda i: (i, 0)),
        pl.BlockSpec((NC, BR, D), lambda i: (0, i, 0)),
        pl.BlockSpec((NC, BR, D), lambda i: (0, i, 0)),
        pl.BlockSpec((1, D), lambda i: (0, 0)),
        pl.BlockSpec((D, D), lambda i: (0, 0)),
    ],
    out_specs=[
        pl.BlockSpec((BR, D), lambda i: (i, 0)),
        pl.BlockSpec((BR, D), lambda i: (i, 0)),
    ],
    out_shape=[
        jax.ShapeDtypeStruct((N, D), jnp.float32),
        jax.ShapeDtypeStruct((N, D), jnp.float32),
    ],
)


def _final_body(y2_ref, q_ref, inv_ref, b2_ref, o_ref):
    s = q_ref[0] + q_ref[1]
    h = inv_ref[...] * s + y2_ref[...] + b2_ref[...]
    n2 = jnp.sum(h * h, axis=1, keepdims=True)
    norm = jnp.sqrt(n2)
    o_ref[...] = h / jnp.maximum(norm, 1e-12)


_final = pl.pallas_call(
    _final_body,
    grid=(N // BR,),
    in_specs=[
        pl.BlockSpec((BR, D), lambda i: (i, 0)),
        pl.BlockSpec((NC, BR, D), lambda i: (0, i, 0)),
        pl.BlockSpec((BR, D), lambda i: (i, 0)),
        pl.BlockSpec((1, D), lambda i: (0, 0)),
    ],
    out_specs=pl.BlockSpec((BR, D), lambda i: (i, 0)),
    out_shape=jax.ShapeDtypeStruct((N, D), jnp.float32),
)


def kernel(x, edge_index, W1, b1, W2, b2):
    row = edge_index[0]
    col = edge_index[1]
    y1 = _mm(x, W1)
    degp, part = _degagg(row, col, y1)
    y2, invb = _mid(y1, part, degp, b1.reshape(1, D), W2)
    qpart = _agg(row, col, y2)
    out = _final(y2, qpart, invb, b2.reshape(1, D))
    return out
